# baseline (speedup 1.0000x reference)
import functools

import jax
import jax.numpy as jnp
from jax import lax
from jax.experimental import pallas as pl
from jax.experimental.pallas import tpu as pltpu
from jax.experimental.pallas import tpu_sc as plsc

_N = 10000
_E = 320000
_D = 128
_NC, _NS = 2, 16
_NW = _NC * _NS
_EPW = _E // _NW
_SB = 1024
_K = 80
_KR = _K // 128
_NSB = 10240 // _SB
_NSUB = _SB // _K
_EPP = 10240
_NP = 10240
_RPT = _NP // _NS
_CR = _NP // _D


_CH = 8


def _make_sc_feat():
    mesh = plsc.VectorSubcoreMesh(core_axis_name="c", subcore_axis_name="s")

    @functools.partial(
        pl.kernel,
        mesh=mesh,
        out_type=jax.ShapeDtypeStruct((_NC * _NP, _D), jnp.float32),
        scratch_types=[
            pltpu.VMEM((_K,), jnp.int32),
            pltpu.VMEM((_K,), jnp.int32),
            pltpu.VMEM((_K, _D), jnp.float32),
            pltpu.VMEM_SHARED((_NP, _D), jnp.float32),
            pltpu.SemaphoreType.DMA,
        ],
        compiler_params=pltpu.CompilerParams(needs_layout_passes=False),
    )
    def sc_feat(h, src1, dst1, zeros, out, sidx, didx, rows, acc, gsem):
        c = lax.axis_index("c")
        s = lax.axis_index("s")
        wid = s * _NC + c
        base = wid * _EPP
        pltpu.sync_copy(zeros.at[pl.ds(s * _RPT, _RPT)],
                        acc.at[pl.ds(s * _RPT, _RPT)])
        plsc.subcore_barrier()

        def body(i, carry):
            off = base + i * _K
            pltpu.sync_copy(src1.at[pl.ds(off, _K)], sidx)
            pltpu.sync_copy(dst1.at[pl.ds(off, _K)], didx)
            pltpu.async_copy(h.at[sidx], rows, gsem).wait()
            pltpu.sync_copy(rows, acc.at[didx], add=True)
            return carry

        lax.fori_loop(0, _EPP // _K, body, 0)
        plsc.subcore_barrier()
        pltpu.sync_copy(acc.at[pl.ds(s * _RPT, _RPT)],
                        out.at[pl.ds(c * _NP + s * _RPT, _RPT)])

    return sc_feat


def _make_sc_cnt():
    mesh = plsc.VectorSubcoreMesh(core_axis_name="c", subcore_axis_name="s")

    @functools.partial(
        pl.kernel,
        mesh=mesh,
        out_type=jax.ShapeDtypeStruct((_NW * _NP,), jnp.float32),
        scratch_types=[
            pltpu.VMEM((_EPP,), jnp.int32),
            pltpu.VMEM((_NP,), jnp.float32),
        ],
        compiler_params=pltpu.CompilerParams(needs_layout_passes=False),
    )
    def sc_cnt(dst1, zcnt, out_cnt, didx, cnt):
        c = lax.axis_index("c")
        s = lax.axis_index("s")
        wid = s * _NC + c
        pltpu.sync_copy(dst1.at[pl.ds(wid * _EPP, _EPP)], didx)
        pltpu.sync_copy(zcnt, cnt)
        ones16 = jnp.ones((16,), jnp.float32)

        def body(i, carry):
            idx = didx[pl.ds(i * 16, 16)]
            plsc.addupdate_scatter(cnt, [idx], ones16)
            return carry

        lax.fori_loop(0, _EPP // 16, body, 0)
        pltpu.sync_copy(cnt, out_cnt.at[pl.ds(wid * _NP, _NP)])

    return sc_cnt


_sc_cache = {}


def _sc_call(name, maker, *args):
    if name not in _sc_cache:
        _sc_cache[name] = maker()
    return _sc_cache[name](*args)


_BR = 2048


def _tc_rcnt_body(cnts, out):
    c = jnp.sum(cnts[...], axis=0)
    out[...] = lax.reciprocal(jnp.maximum(c, 1.0))


_tc_rcnt = pl.pallas_call(
    _tc_rcnt_body,
    out_shape=jax.ShapeDtypeStruct((_CR, _D), jnp.float32),
)


def _mean_and_out(pa, pb, rc, hext, wl, bl, wr):
    p = pa[0] + pb[0]
    mean = p * rc[...]
    h = hext[...]
    o = jnp.dot(mean, wl[...], preferred_element_type=jnp.float32) + bl[...]
    o = o + jnp.dot(h, wr[...], preferred_element_type=jnp.float32)
    nrm2 = jnp.sum(o * o, axis=1, keepdims=True)
    return h, o * lax.rsqrt(jnp.maximum(nrm2, 1e-24))


def _tc_layer1_body(pa, pb, rc, hext, wl, bl, wr, out):
    _, o = _mean_and_out(pa, pb, rc, hext, wl, bl, wr)
    out[...] = jnp.maximum(o, 0.0)


def _tc_layer2_body(pa, pb, rc, hext, wl, bl, wr, wlin, blin, out):
    h, o = _mean_and_out(pa, pb, rc, hext, wl, bl, wr)
    h2 = jnp.maximum(o + h, 0.0)
    out[...] = jnp.dot(h2, wlin[...], preferred_element_type=jnp.float32) + blin[...]


_full = lambda i: (0, 0)

_layer_specs = [
    pl.BlockSpec((1, _BR, _D), lambda i: (0, i, 0)),
    pl.BlockSpec((1, _BR, _D), lambda i: (1, i, 0)),
    pl.BlockSpec((_BR, 1), lambda i: (i, 0)),
    pl.BlockSpec((_BR, _D), lambda i: (i, 0)),
    pl.BlockSpec((_D, _D), _full),
    pl.BlockSpec((1, _D), _full),
    pl.BlockSpec((_D, _D), _full),
]

_tc_layer1 = pl.pallas_call(
    _tc_layer1_body,
    grid=(_NP // _BR,),
    in_specs=list(_layer_specs),
    out_specs=pl.BlockSpec((_BR, _D), lambda i: (i, 0)),
    out_shape=jax.ShapeDtypeStruct((_NP, _D), jnp.float32),
)

_tc_layer2 = pl.pallas_call(
    _tc_layer2_body,
    grid=(_NP // _BR,),
    in_specs=list(_layer_specs) + [
        pl.BlockSpec((_D, 2), _full),
        pl.BlockSpec((1, 2), _full),
    ],
    out_specs=pl.BlockSpec((_BR, 2), lambda i: (i, 0)),
    out_shape=jax.ShapeDtypeStruct((_NP, 2), jnp.float32),
)


def kernel(x, edge_index, W1_l, b1_l, W1_r, W2_l, b2_l, W2_r, W_lin, b_lin):
    src1 = jnp.pad(edge_index[0].reshape(_NW, _EPW),
                   ((0, 0), (0, _EPP - _EPW))).reshape(_NW * _EPP)
    padrows = jnp.broadcast_to(_N + jnp.arange(_EPP - _EPW, dtype=jnp.int32),
                               (_NW, _EPP - _EPW))
    dst1 = jnp.concatenate(
        [edge_index[1].reshape(_NW, _EPW), padrows], axis=1).reshape(_NW * _EPP)
    zeros = jnp.zeros((_NP, _D), jnp.float32)
    zcnt = jnp.zeros((_NP,), jnp.float32)
    x_pad = jnp.concatenate([x, jnp.zeros((_NP - _N, _D), jnp.float32)], axis=0)

    p1 = _sc_call("feat", _make_sc_feat, x_pad, src1, dst1, zeros)
    c1 = _sc_call("cnt", _make_sc_cnt, dst1, zcnt)
    p1 = p1.reshape(_NC, _NP, _D)
    rc = _tc_rcnt(c1.reshape(_NW, _CR, _D)).reshape(_NP, 1)
    h1 = _tc_layer1(p1, p1, rc, x_pad, W1_l, b1_l.reshape(1, _D), W1_r)
    p2 = _sc_call("feat", _make_sc_feat, h1, src1, dst1, zeros)
    p2 = p2.reshape(_NC, _NP, _D)
    out = _tc_layer2(p2, p2, rc, h1, W2_l, b2_l.reshape(1, _D), W2_r,
                     W_lin, b_lin.reshape(1, 2))
    return out[:_N]


# speedup vs baseline: 2.6315x; 2.6315x over previous
import functools

import jax
import jax.numpy as jnp
from jax import lax
from jax.experimental import pallas as pl
from jax.experimental.pallas import tpu as pltpu
from jax.experimental.pallas import tpu_sc as plsc

_N = 10000
_E = 320000
_D = 128
_NC, _NS = 2, 16
_NW = _NC * _NS
_EPW = _E // _NW
_SB = 1024
_K = 200
_KR = _K // 128
_NSB = 10240 // _SB
_NSUB = _SB // _K
_EPP = 10240
_NP = 10240
_RPT = _NP // _NS
_CR = _NP // _D


_CH = 8


def _make_sc_feat():
    mesh = plsc.VectorSubcoreMesh(core_axis_name="c", subcore_axis_name="s")

    @functools.partial(
        pl.kernel,
        mesh=mesh,
        out_type=jax.ShapeDtypeStruct((_NC * _NP, _D), jnp.float32),
        scratch_types=[
            pltpu.VMEM((_K,), jnp.int32),
            pltpu.VMEM((_K,), jnp.int32),
            pltpu.VMEM((_K, _D), jnp.float32),
            pltpu.VMEM_SHARED((_NP, _D), jnp.float32),
            pltpu.SemaphoreType.DMA,
        ],
        compiler_params=pltpu.CompilerParams(needs_layout_passes=False),
    )
    def sc_feat(h, src1, dst1, zeros, out, sidx, didx, rows, acc, gsem):
        c = lax.axis_index("c")
        s = lax.axis_index("s")
        wid = s * _NC + c
        base = wid * _EPW
        pltpu.sync_copy(zeros.at[pl.ds(s * _RPT, _RPT)],
                        acc.at[pl.ds(s * _RPT, _RPT)])
        plsc.subcore_barrier()

        def body(i, carry):
            off = base + i * _K
            pltpu.sync_copy(src1.at[pl.ds(off, _K)], sidx)
            pltpu.sync_copy(dst1.at[pl.ds(off, _K)], didx)
            pltpu.async_copy(h.at[sidx], rows, gsem).wait()
            pltpu.sync_copy(rows, acc.at[didx], add=True)
            return carry

        lax.fori_loop(0, _EPW // _K, body, 0)
        plsc.subcore_barrier()
        pltpu.sync_copy(acc.at[pl.ds(s * _RPT, _RPT)],
                        out.at[pl.ds(c * _NP + s * _RPT, _RPT)])

    return sc_feat


def _make_sc_cnt():
    mesh = plsc.VectorSubcoreMesh(core_axis_name="c", subcore_axis_name="s")

    @functools.partial(
        pl.kernel,
        mesh=mesh,
        out_type=jax.ShapeDtypeStruct((_NW * _NP,), jnp.float32),
        scratch_types=[
            pltpu.VMEM((_EPW,), jnp.int32),
            pltpu.VMEM((_NP,), jnp.float32),
        ],
        compiler_params=pltpu.CompilerParams(needs_layout_passes=False),
    )
    def sc_cnt(dst1, zcnt, out_cnt, didx, cnt):
        c = lax.axis_index("c")
        s = lax.axis_index("s")
        wid = s * _NC + c
        pltpu.sync_copy(dst1.at[pl.ds(wid * _EPW, _EPW)], didx)
        pltpu.sync_copy(zcnt, cnt)
        ones16 = jnp.ones((16,), jnp.float32)

        def body(i, carry):
            idx = didx[pl.ds(i * 16, 16)]
            plsc.addupdate_scatter(cnt, [idx], ones16)
            return carry

        lax.fori_loop(0, _EPW // 16, body, 0)
        pltpu.sync_copy(cnt, out_cnt.at[pl.ds(wid * _NP, _NP)])

    return sc_cnt


_sc_cache = {}


def _sc_call(name, maker, *args):
    if name not in _sc_cache:
        _sc_cache[name] = maker()
    return _sc_cache[name](*args)


_BR = 2048


def _tc_rcnt_body(cnts, out):
    c = jnp.sum(cnts[...], axis=0)
    out[...] = lax.reciprocal(jnp.maximum(c, 1.0))


_tc_rcnt = pl.pallas_call(
    _tc_rcnt_body,
    out_shape=jax.ShapeDtypeStruct((_CR, _D), jnp.float32),
)


def _mean_and_out(pa, pb, rc, hext, wl, bl, wr):
    p = pa[0] + pb[0]
    mean = p * rc[...]
    h = hext[...]
    o = jnp.dot(mean, wl[...], preferred_element_type=jnp.float32) + bl[...]
    o = o + jnp.dot(h, wr[...], preferred_element_type=jnp.float32)
    nrm2 = jnp.sum(o * o, axis=1, keepdims=True)
    return h, o * lax.rsqrt(jnp.maximum(nrm2, 1e-24))


def _tc_layer1_body(pa, pb, rc, hext, wl, bl, wr, out):
    _, o = _mean_and_out(pa, pb, rc, hext, wl, bl, wr)
    out[...] = jnp.maximum(o, 0.0)


def _tc_layer2_body(pa, pb, rc, hext, wl, bl, wr, wlin, blin, out):
    h, o = _mean_and_out(pa, pb, rc, hext, wl, bl, wr)
    h2 = jnp.maximum(o + h, 0.0)
    out[...] = jnp.dot(h2, wlin[...], preferred_element_type=jnp.float32) + blin[...]


_full = lambda i: (0, 0)

_layer_specs = [
    pl.BlockSpec((1, _BR, _D), lambda i: (0, i, 0)),
    pl.BlockSpec((1, _BR, _D), lambda i: (1, i, 0)),
    pl.BlockSpec((_BR, 1), lambda i: (i, 0)),
    pl.BlockSpec((_BR, _D), lambda i: (i, 0)),
    pl.BlockSpec((_D, _D), _full),
    pl.BlockSpec((1, _D), _full),
    pl.BlockSpec((_D, _D), _full),
]

_tc_layer1 = pl.pallas_call(
    _tc_layer1_body,
    grid=(_NP // _BR,),
    in_specs=list(_layer_specs),
    out_specs=pl.BlockSpec((_BR, _D), lambda i: (i, 0)),
    out_shape=jax.ShapeDtypeStruct((_NP, _D), jnp.float32),
)

_tc_layer2 = pl.pallas_call(
    _tc_layer2_body,
    grid=(_NP // _BR,),
    in_specs=list(_layer_specs) + [
        pl.BlockSpec((_D, 2), _full),
        pl.BlockSpec((1, 2), _full),
    ],
    out_specs=pl.BlockSpec((_BR, 2), lambda i: (i, 0)),
    out_shape=jax.ShapeDtypeStruct((_NP, 2), jnp.float32),
)


def kernel(x, edge_index, W1_l, b1_l, W1_r, W2_l, b2_l, W2_r, W_lin, b_lin):
    src1 = edge_index[0]
    dst1 = edge_index[1]
    zeros = jnp.zeros((_NP, _D), jnp.float32)
    zcnt = jnp.zeros((_NP,), jnp.float32)
    x_pad = jnp.concatenate([x, jnp.zeros((_NP - _N, _D), jnp.float32)], axis=0)

    p1 = _sc_call("feat", _make_sc_feat, x_pad, src1, dst1, zeros)
    c1 = _sc_call("cnt", _make_sc_cnt, dst1, zcnt)
    p1 = p1.reshape(_NC, _NP, _D)
    rc = _tc_rcnt(c1.reshape(_NW, _CR, _D)).reshape(_NP, 1)
    h1 = _tc_layer1(p1, p1, rc, x_pad, W1_l, b1_l.reshape(1, _D), W1_r)
    p2 = _sc_call("feat", _make_sc_feat, h1, src1, dst1, zeros)
    p2 = p2.reshape(_NC, _NP, _D)
    out = _tc_layer2(p2, p2, rc, h1, W2_l, b2_l.reshape(1, _D), W2_r,
                     W_lin, b_lin.reshape(1, 2))
    return out[:_N]


# speedup vs baseline: 2.9409x; 1.1175x over previous
import functools

import jax
import jax.numpy as jnp
from jax import lax
from jax.experimental import pallas as pl
from jax.experimental.pallas import tpu as pltpu
from jax.experimental.pallas import tpu_sc as plsc

_N = 10000
_E = 320000
_D = 128
_NC, _NS = 2, 16
_NW = _NC * _NS
_EPW = _E // _NW
_SB = 1024
_K = 320
_KT = _EPW - (_EPW // _K) * _K
_KR = _K // 128
_NSB = 10240 // _SB
_NSUB = _SB // _K
_EPP = 10240
_NP = 10240
_RPT = _NP // _NS
_CR = _NP // _D


_CH = 8


def _make_sc_feat():
    mesh = plsc.VectorSubcoreMesh(core_axis_name="c", subcore_axis_name="s")

    @functools.partial(
        pl.kernel,
        mesh=mesh,
        out_type=jax.ShapeDtypeStruct((_NC * _NP, _D), jnp.float32),
        scratch_types=[
            pltpu.VMEM((_K,), jnp.int32),
            pltpu.VMEM((_K,), jnp.int32),
            pltpu.VMEM((_KT,), jnp.int32),
            pltpu.VMEM((_KT,), jnp.int32),
            pltpu.VMEM((_K, _D), jnp.float32),
            pltpu.VMEM_SHARED((_NP, _D), jnp.float32),
            pltpu.SemaphoreType.DMA,
        ],
        compiler_params=pltpu.CompilerParams(needs_layout_passes=False),
    )
    def sc_feat(h, src1, dst1, zeros, out,
                sidx, didx, sidx_t, didx_t, rows, acc, gsem):
        c = lax.axis_index("c")
        s = lax.axis_index("s")
        wid = s * _NC + c
        base = wid * _EPW
        pltpu.sync_copy(zeros.at[pl.ds(s * _RPT, _RPT)],
                        acc.at[pl.ds(s * _RPT, _RPT)])
        plsc.subcore_barrier()

        def body(i, carry):
            off = base + i * _K
            pltpu.sync_copy(src1.at[pl.ds(off, _K)], sidx)
            pltpu.sync_copy(dst1.at[pl.ds(off, _K)], didx)
            pltpu.async_copy(h.at[sidx], rows, gsem).wait()
            pltpu.sync_copy(rows, acc.at[didx], add=True)
            return carry

        lax.fori_loop(0, _EPW // _K, body, 0)
        off = base + (_EPW // _K) * _K
        pltpu.sync_copy(src1.at[pl.ds(off, _KT)], sidx_t)
        pltpu.sync_copy(dst1.at[pl.ds(off, _KT)], didx_t)
        rows_t = rows.at[pl.ds(0, _KT)]
        pltpu.async_copy(h.at[sidx_t], rows_t, gsem).wait()
        pltpu.sync_copy(rows_t, acc.at[didx_t], add=True)
        plsc.subcore_barrier()
        pltpu.sync_copy(acc.at[pl.ds(s * _RPT, _RPT)],
                        out.at[pl.ds(c * _NP + s * _RPT, _RPT)])

    return sc_feat


def _make_sc_cnt():
    mesh = plsc.VectorSubcoreMesh(core_axis_name="c", subcore_axis_name="s")

    @functools.partial(
        pl.kernel,
        mesh=mesh,
        out_type=jax.ShapeDtypeStruct((_NW * _NP,), jnp.float32),
        scratch_types=[
            pltpu.VMEM((_EPW,), jnp.int32),
            pltpu.VMEM((_NP,), jnp.float32),
        ],
        compiler_params=pltpu.CompilerParams(needs_layout_passes=False),
    )
    def sc_cnt(dst1, zcnt, out_cnt, didx, cnt):
        c = lax.axis_index("c")
        s = lax.axis_index("s")
        wid = s * _NC + c
        pltpu.sync_copy(dst1.at[pl.ds(wid * _EPW, _EPW)], didx)
        pltpu.sync_copy(zcnt, cnt)
        ones16 = jnp.ones((16,), jnp.float32)

        def body(i, carry):
            idx = didx[pl.ds(i * 16, 16)]
            plsc.addupdate_scatter(cnt, [idx], ones16)
            return carry

        lax.fori_loop(0, _EPW // 16, body, 0)
        pltpu.sync_copy(cnt, out_cnt.at[pl.ds(wid * _NP, _NP)])

    return sc_cnt


_sc_cache = {}


def _sc_call(name, maker, *args):
    if name not in _sc_cache:
        _sc_cache[name] = maker()
    return _sc_cache[name](*args)


_BR = 2048


def _tc_rcnt_body(cnts, out):
    c = jnp.sum(cnts[...], axis=0)
    out[...] = lax.reciprocal(jnp.maximum(c, 1.0))


_tc_rcnt = pl.pallas_call(
    _tc_rcnt_body,
    out_shape=jax.ShapeDtypeStruct((_CR, _D), jnp.float32),
)


def _mean_and_out(pa, pb, rc, hext, wl, bl, wr):
    p = pa[0] + pb[0]
    mean = p * rc[...]
    h = hext[...]
    o = jnp.dot(mean, wl[...], preferred_element_type=jnp.float32) + bl[...]
    o = o + jnp.dot(h, wr[...], preferred_element_type=jnp.float32)
    nrm2 = jnp.sum(o * o, axis=1, keepdims=True)
    return h, o * lax.rsqrt(jnp.maximum(nrm2, 1e-24))


def _tc_layer1_body(pa, pb, rc, hext, wl, bl, wr, out):
    _, o = _mean_and_out(pa, pb, rc, hext, wl, bl, wr)
    out[...] = jnp.maximum(o, 0.0)


def _tc_layer2_body(pa, pb, rc, hext, wl, bl, wr, wlin, blin, out):
    h, o = _mean_and_out(pa, pb, rc, hext, wl, bl, wr)
    h2 = jnp.maximum(o + h, 0.0)
    out[...] = jnp.dot(h2, wlin[...], preferred_element_type=jnp.float32) + blin[...]


_full = lambda i: (0, 0)

_layer_specs = [
    pl.BlockSpec((1, _BR, _D), lambda i: (0, i, 0)),
    pl.BlockSpec((1, _BR, _D), lambda i: (1, i, 0)),
    pl.BlockSpec((_BR, 1), lambda i: (i, 0)),
    pl.BlockSpec((_BR, _D), lambda i: (i, 0)),
    pl.BlockSpec((_D, _D), _full),
    pl.BlockSpec((1, _D), _full),
    pl.BlockSpec((_D, _D), _full),
]

_tc_layer1 = pl.pallas_call(
    _tc_layer1_body,
    grid=(_NP // _BR,),
    in_specs=list(_layer_specs),
    out_specs=pl.BlockSpec((_BR, _D), lambda i: (i, 0)),
    out_shape=jax.ShapeDtypeStruct((_NP, _D), jnp.float32),
)

_tc_layer2 = pl.pallas_call(
    _tc_layer2_body,
    grid=(_NP // _BR,),
    in_specs=list(_layer_specs) + [
        pl.BlockSpec((_D, 2), _full),
        pl.BlockSpec((1, 2), _full),
    ],
    out_specs=pl.BlockSpec((_BR, 2), lambda i: (i, 0)),
    out_shape=jax.ShapeDtypeStruct((_NP, 2), jnp.float32),
)


def kernel(x, edge_index, W1_l, b1_l, W1_r, W2_l, b2_l, W2_r, W_lin, b_lin):
    src1 = edge_index[0]
    dst1 = edge_index[1]
    zeros = jnp.zeros((_NP, _D), jnp.float32)
    zcnt = jnp.zeros((_NP,), jnp.float32)
    x_pad = jnp.concatenate([x, jnp.zeros((_NP - _N, _D), jnp.float32)], axis=0)

    p1 = _sc_call("feat", _make_sc_feat, x_pad, src1, dst1, zeros)
    c1 = _sc_call("cnt", _make_sc_cnt, dst1, zcnt)
    p1 = p1.reshape(_NC, _NP, _D)
    rc = _tc_rcnt(c1.reshape(_NW, _CR, _D)).reshape(_NP, 1)
    h1 = _tc_layer1(p1, p1, rc, x_pad, W1_l, b1_l.reshape(1, _D), W1_r)
    p2 = _sc_call("feat", _make_sc_feat, h1, src1, dst1, zeros)
    p2 = p2.reshape(_NC, _NP, _D)
    out = _tc_layer2(p2, p2, rc, h1, W2_l, b2_l.reshape(1, _D), W2_r,
                     W_lin, b_lin.reshape(1, 2))
    return out[:_N]


# speedup vs baseline: 2.9733x; 1.0110x over previous
import functools

import jax
import jax.numpy as jnp
from jax import lax
from jax.experimental import pallas as pl
from jax.experimental.pallas import tpu as pltpu
from jax.experimental.pallas import tpu_sc as plsc

_N = 10000
_E = 320000
_D = 128
_NC, _NS = 2, 16
_NW = _NC * _NS
_EPW = _E // _NW
_SB = 1024
_K = 160
_KT = _EPW - (_EPW // _K) * _K
_KR = _K // 128
_NSB = 10240 // _SB
_NSUB = _SB // _K
_EPP = 10240
_NP = 10240
_RPT = _NP // _NS
_CR = _NP // _D


_CH = 8


def _make_sc_feat():
    mesh = plsc.VectorSubcoreMesh(core_axis_name="c", subcore_axis_name="s")

    @functools.partial(
        pl.kernel,
        mesh=mesh,
        out_type=jax.ShapeDtypeStruct((_NC * _NP, _D), jnp.float32),
        scratch_types=[
            pltpu.VMEM((_K,), jnp.int32),
            pltpu.VMEM((_K,), jnp.int32),
            pltpu.VMEM((_K,), jnp.int32),
            pltpu.VMEM((_K,), jnp.int32),
            pltpu.VMEM((_KT,), jnp.int32),
            pltpu.VMEM((_KT,), jnp.int32),
            pltpu.VMEM((_K, _D), jnp.float32),
            pltpu.VMEM((_K, _D), jnp.float32),
            pltpu.VMEM_SHARED((_NP, _D), jnp.float32),
            pltpu.SemaphoreType.DMA,
            pltpu.SemaphoreType.DMA,
        ],
        compiler_params=pltpu.CompilerParams(needs_layout_passes=False),
    )
    def sc_feat(h, src1, dst1, zeros, out,
                sidx0, didx0, sidx1, didx1, sidx_t, didx_t,
                rows0, rows1, acc, gsem, ssem):
        c = lax.axis_index("c")
        s = lax.axis_index("s")
        wid = s * _NC + c
        base = wid * _EPW
        pltpu.sync_copy(zeros.at[pl.ds(s * _RPT, _RPT)],
                        acc.at[pl.ds(s * _RPT, _RPT)])
        plsc.subcore_barrier()

        slots = ((sidx0, didx0, rows0), (sidx1, didx1, rows1))

        def load_and_gather(i, sl):
            sidx, didx, rows = slots[sl]
            off = base + i * _K
            pltpu.sync_copy(src1.at[pl.ds(off, _K)], sidx)
            pltpu.sync_copy(dst1.at[pl.ds(off, _K)], didx)
            pltpu.async_copy(h.at[sidx], rows, gsem).wait()

        def scat_start(sl):
            _, didx, rows = slots[sl]
            pltpu.async_copy(rows, acc.at[didx], ssem, add=True)

        def scat_wait(sl):
            _, didx, rows = slots[sl]
            pltpu.make_async_copy(rows, acc.at[didx], ssem).wait()

        load_and_gather(0, 0)
        scat_start(0)
        load_and_gather(1, 1)
        scat_start(1)

        def body(ii, carry):
            i0 = 2 * ii
            scat_wait(0)
            load_and_gather(i0, 0)
            scat_start(0)
            scat_wait(1)
            load_and_gather(i0 + 1, 1)
            scat_start(1)
            return carry

        lax.fori_loop(1, _EPW // _K // 2, body, 0)
        scat_wait(0)
        scat_wait(1)
        off = base + (_EPW // _K) * _K
        pltpu.sync_copy(src1.at[pl.ds(off, _KT)], sidx_t)
        pltpu.sync_copy(dst1.at[pl.ds(off, _KT)], didx_t)
        rows_t = rows0.at[pl.ds(0, _KT)]
        pltpu.async_copy(h.at[sidx_t], rows_t, gsem).wait()
        pltpu.sync_copy(rows_t, acc.at[didx_t], add=True)
        plsc.subcore_barrier()
        pltpu.sync_copy(acc.at[pl.ds(s * _RPT, _RPT)],
                        out.at[pl.ds(c * _NP + s * _RPT, _RPT)])

    return sc_feat


def _make_sc_cnt():
    mesh = plsc.VectorSubcoreMesh(core_axis_name="c", subcore_axis_name="s")

    @functools.partial(
        pl.kernel,
        mesh=mesh,
        out_type=jax.ShapeDtypeStruct((_NW * _NP,), jnp.float32),
        scratch_types=[
            pltpu.VMEM((_EPW,), jnp.int32),
            pltpu.VMEM((_NP,), jnp.float32),
        ],
        compiler_params=pltpu.CompilerParams(needs_layout_passes=False),
    )
    def sc_cnt(dst1, zcnt, out_cnt, didx, cnt):
        c = lax.axis_index("c")
        s = lax.axis_index("s")
        wid = s * _NC + c
        pltpu.sync_copy(dst1.at[pl.ds(wid * _EPW, _EPW)], didx)
        pltpu.sync_copy(zcnt, cnt)
        ones16 = jnp.ones((16,), jnp.float32)

        def body(i, carry):
            idx = didx[pl.ds(i * 16, 16)]
            plsc.addupdate_scatter(cnt, [idx], ones16)
            return carry

        lax.fori_loop(0, _EPW // 16, body, 0)
        pltpu.sync_copy(cnt, out_cnt.at[pl.ds(wid * _NP, _NP)])

    return sc_cnt


_sc_cache = {}


def _sc_call(name, maker, *args):
    if name not in _sc_cache:
        _sc_cache[name] = maker()
    return _sc_cache[name](*args)


_BR = 2048


def _tc_rcnt_body(cnts, out):
    c = jnp.sum(cnts[...], axis=0)
    out[...] = lax.reciprocal(jnp.maximum(c, 1.0))


_tc_rcnt = pl.pallas_call(
    _tc_rcnt_body,
    out_shape=jax.ShapeDtypeStruct((_CR, _D), jnp.float32),
)


def _mean_and_out(pa, pb, rc, hext, wl, bl, wr):
    p = pa[0] + pb[0]
    mean = p * rc[...]
    h = hext[...]
    o = jnp.dot(mean, wl[...], preferred_element_type=jnp.float32) + bl[...]
    o = o + jnp.dot(h, wr[...], preferred_element_type=jnp.float32)
    nrm2 = jnp.sum(o * o, axis=1, keepdims=True)
    return h, o * lax.rsqrt(jnp.maximum(nrm2, 1e-24))


def _tc_layer1_body(pa, pb, rc, hext, wl, bl, wr, out):
    _, o = _mean_and_out(pa, pb, rc, hext, wl, bl, wr)
    out[...] = jnp.maximum(o, 0.0)


def _tc_layer2_body(pa, pb, rc, hext, wl, bl, wr, wlin, blin, out):
    h, o = _mean_and_out(pa, pb, rc, hext, wl, bl, wr)
    h2 = jnp.maximum(o + h, 0.0)
    out[...] = jnp.dot(h2, wlin[...], preferred_element_type=jnp.float32) + blin[...]


_full = lambda i: (0, 0)

_layer_specs = [
    pl.BlockSpec((1, _BR, _D), lambda i: (0, i, 0)),
    pl.BlockSpec((1, _BR, _D), lambda i: (1, i, 0)),
    pl.BlockSpec((_BR, 1), lambda i: (i, 0)),
    pl.BlockSpec((_BR, _D), lambda i: (i, 0)),
    pl.BlockSpec((_D, _D), _full),
    pl.BlockSpec((1, _D), _full),
    pl.BlockSpec((_D, _D), _full),
]

_tc_layer1 = pl.pallas_call(
    _tc_layer1_body,
    grid=(_NP // _BR,),
    in_specs=list(_layer_specs),
    out_specs=pl.BlockSpec((_BR, _D), lambda i: (i, 0)),
    out_shape=jax.ShapeDtypeStruct((_NP, _D), jnp.float32),
)

_tc_layer2 = pl.pallas_call(
    _tc_layer2_body,
    grid=(_NP // _BR,),
    in_specs=list(_layer_specs) + [
        pl.BlockSpec((_D, 2), _full),
        pl.BlockSpec((1, 2), _full),
    ],
    out_specs=pl.BlockSpec((_BR, 2), lambda i: (i, 0)),
    out_shape=jax.ShapeDtypeStruct((_NP, 2), jnp.float32),
)


def kernel(x, edge_index, W1_l, b1_l, W1_r, W2_l, b2_l, W2_r, W_lin, b_lin):
    src1 = edge_index[0]
    dst1 = edge_index[1]
    zeros = jnp.zeros((_NP, _D), jnp.float32)
    zcnt = jnp.zeros((_NP,), jnp.float32)
    x_pad = jnp.concatenate([x, jnp.zeros((_NP - _N, _D), jnp.float32)], axis=0)

    p1 = _sc_call("feat", _make_sc_feat, x_pad, src1, dst1, zeros)
    c1 = _sc_call("cnt", _make_sc_cnt, dst1, zcnt)
    p1 = p1.reshape(_NC, _NP, _D)
    rc = _tc_rcnt(c1.reshape(_NW, _CR, _D)).reshape(_NP, 1)
    h1 = _tc_layer1(p1, p1, rc, x_pad, W1_l, b1_l.reshape(1, _D), W1_r)
    p2 = _sc_call("feat", _make_sc_feat, h1, src1, dst1, zeros)
    p2 = p2.reshape(_NC, _NP, _D)
    out = _tc_layer2(p2, p2, rc, h1, W2_l, b2_l.reshape(1, _D), W2_r,
                     W_lin, b_lin.reshape(1, 2))
    return out[:_N]
